# Initial kernel scaffold; baseline (speedup 1.0000x reference)
import jax
import jax.numpy as jnp
from jax.experimental import pallas as pl


def kernel(x, expand_w, expand_b, pe_norm_g, pe_norm_b, norm1_g, norm1_b, qkv_w, qkv_b, proj_w, proj_b, rel_bias, norm2_g, norm2_b, mlp_w1, mlp_b1, mlp_w2, mlp_b2):
    raise NotImplementedError("write your pallas kernel here")


# speedup vs baseline: 2.9882x; 2.9882x over previous
import functools

import jax
import jax.numpy as jnp
import numpy as np
from jax.experimental import pallas as pl
from jax.experimental.pallas import tpu as pltpu

WS = 7
HEADS = 8
HEAD_DIM = 256
INNER = HEADS * HEAD_DIM
DIM = 256
SCALE = HEAD_DIM ** -0.5
N = WS * WS
NPAD = 56
WIN_PER_STEP = 8
NEG = -1e9


def _rel_index_np():
    coords = np.stack(np.meshgrid(np.arange(WS), np.arange(WS), indexing='ij')).reshape(2, -1)
    rel = (coords[:, :, None] - coords[:, None, :]).transpose(1, 2, 0)
    rel[..., 0] += WS - 1
    rel[..., 1] += WS - 1
    rel[..., 0] *= 2 * WS - 1
    return rel.sum(-1)


_REL_IDX = _rel_index_np()


def _shift_mask_np(H, W):
    shift = WS // 2
    img = np.zeros((H, W))
    cnt = 0
    for hs in (slice(0, -WS), slice(-WS, -shift), slice(-shift, None)):
        for ws_ in (slice(0, -WS), slice(-WS, -shift), slice(-shift, None)):
            img[hs, ws_] = cnt
            cnt += 1
    mw = img.reshape(H // WS, WS, W // WS, WS).transpose(0, 2, 1, 3).reshape(-1, N)
    diff = mw[:, None, :] - mw[:, :, None]
    return np.where(diff != 0, -100.0, 0.0).astype(np.float32)


_SHIFT_MASK = _shift_mask_np(56, 56)


def _expand_kernel(x_ref, w_ref, b_ref, g_ref, bn_ref, o_ref):
    y = jnp.dot(x_ref[...], w_ref[...], preferred_element_type=jnp.float32)
    y = y + b_ref[...]
    g = g_ref[...]
    bn = bn_ref[...]
    for j in range(4):
        c = y[:, j * DIM:(j + 1) * DIM]
        m = jnp.mean(c, axis=-1, keepdims=True)
        d = c - m
        v = jnp.mean(d * d, axis=-1, keepdims=True)
        o_ref[:, j * DIM:(j + 1) * DIM] = d * jax.lax.rsqrt(v + 1e-5) * g + bn


def _swin_kernel(x_ref, bias_ref, n1g_ref, n1b_ref, qkvw_ref, qkvb_ref,
                 pw_ref, pb_ref, n2g_ref, n2b_ref, w1_ref, b1_ref,
                 w2_ref, b2_ref, o_ref):
    M = WIN_PER_STEP * NPAD
    x = x_ref[...].reshape(M, DIM)

    m = jnp.mean(x, axis=-1, keepdims=True)
    d = x - m
    v = jnp.mean(d * d, axis=-1, keepdims=True)
    y = d * jax.lax.rsqrt(v + 1e-5) * n1g_ref[...] + n1b_ref[...]

    qkv = jnp.dot(y, qkvw_ref[...], preferred_element_type=jnp.float32)
    qkv = qkv + qkvb_ref[...]

    o_rows = []
    for w in range(WIN_PER_STEP):
        r0 = w * NPAD
        o_heads = []
        for h in range(HEADS):
            q = qkv[r0:r0 + NPAD, h * HEAD_DIM:(h + 1) * HEAD_DIM]
            k = qkv[r0:r0 + NPAD, INNER + h * HEAD_DIM:INNER + (h + 1) * HEAD_DIM]
            vv = qkv[r0:r0 + NPAD, 2 * INNER + h * HEAD_DIM:2 * INNER + (h + 1) * HEAD_DIM]
            s = jax.lax.dot_general(q, k, (((1,), (1,)), ((), ())),
                                    preferred_element_type=jnp.float32)
            s = s * SCALE + bias_ref[w, h]
            mx = jnp.max(s, axis=-1, keepdims=True)
            e = jnp.exp(s - mx)
            p = e / jnp.sum(e, axis=-1, keepdims=True)
            o_heads.append(jnp.dot(p, vv, preferred_element_type=jnp.float32))
        o_rows.append(jnp.concatenate(o_heads, axis=1))
    o = jnp.concatenate(o_rows, axis=0)

    o = jnp.dot(o, pw_ref[...], preferred_element_type=jnp.float32) + pb_ref[...]
    x1 = x + o

    m2 = jnp.mean(x1, axis=-1, keepdims=True)
    d2 = x1 - m2
    v2 = jnp.mean(d2 * d2, axis=-1, keepdims=True)
    z = d2 * jax.lax.rsqrt(v2 + 1e-5) * n2g_ref[...] + n2b_ref[...]
    hmid = jnp.dot(z, w1_ref[...], preferred_element_type=jnp.float32) + b1_ref[...]
    hmid = jax.nn.gelu(hmid)
    z2 = jnp.dot(hmid, w2_ref[...], preferred_element_type=jnp.float32) + b2_ref[...]
    o_ref[...] = (x1 + z2).reshape(WIN_PER_STEP, NPAD, DIM)


def _swin_block(xw, bias, n1g, n1b, qkvw, qkvb, pw, pb, n2g, n2b, w1, b1, w2, b2):
    nwin = xw.shape[0]
    grid = (nwin // WIN_PER_STEP,)
    G = bias.shape[0] // WIN_PER_STEP
    row = lambda s: (s, 0, 0)
    full2 = lambda s: (0, 0)
    return pl.pallas_call(
        _swin_kernel,
        grid=grid,
        in_specs=[
            pl.BlockSpec((WIN_PER_STEP, NPAD, DIM), row),
            pl.BlockSpec((WIN_PER_STEP, HEADS, NPAD, NPAD),
                         lambda s: (s % G, 0, 0, 0)),
            pl.BlockSpec((1, DIM), full2),
            pl.BlockSpec((1, DIM), full2),
            pl.BlockSpec((DIM, 3 * INNER), full2),
            pl.BlockSpec((1, 3 * INNER), full2),
            pl.BlockSpec((INNER, DIM), full2),
            pl.BlockSpec((1, DIM), full2),
            pl.BlockSpec((1, DIM), full2),
            pl.BlockSpec((1, DIM), full2),
            pl.BlockSpec((DIM, 4 * DIM), full2),
            pl.BlockSpec((1, 4 * DIM), full2),
            pl.BlockSpec((4 * DIM, DIM), full2),
            pl.BlockSpec((1, DIM), full2),
        ],
        out_specs=pl.BlockSpec((WIN_PER_STEP, NPAD, DIM), row),
        out_shape=jax.ShapeDtypeStruct((nwin, NPAD, DIM), jnp.float32),
        compiler_params=pltpu.CompilerParams(
            dimension_semantics=("parallel",),
            vmem_limit_bytes=100 * 1024 * 1024,
        ),
    )(xw, bias, n1g, n1b, qkvw, qkvb, pw, pb, n2g, n2b, w1, b1, w2, b2)


def _windows_pad(x):
    B, H, W, C = x.shape
    xw = x.reshape(B, H // WS, WS, W // WS, WS, C).transpose(0, 1, 3, 2, 4, 5)
    xw = xw.reshape(-1, N, C)
    return jnp.pad(xw, ((0, 0), (0, NPAD - N), (0, 0)))


def _unwindows(xw, B, H, W):
    C = xw.shape[-1]
    xw = xw[:, :N, :]
    xw = xw.reshape(B, H // WS, W // WS, WS, WS, C).transpose(0, 1, 3, 2, 4, 5)
    return xw.reshape(B, H, W, C)


@jax.jit
def kernel(x, expand_w, expand_b, pe_norm_g, pe_norm_b, norm1_g, norm1_b,
           qkv_w, qkv_b, proj_w, proj_b, rel_bias, norm2_g, norm2_b,
           mlp_w1, mlp_b1, mlp_w2, mlp_b2):
    B, h, w, Cin = x.shape
    H, W = h * 2, w * 2
    tokens = B * h * w

    xf = x.reshape(tokens, Cin)
    MB = 392
    y = pl.pallas_call(
        _expand_kernel,
        grid=(tokens // MB,),
        in_specs=[
            pl.BlockSpec((MB, Cin), lambda s: (s, 0)),
            pl.BlockSpec((Cin, 4 * DIM), lambda s: (0, 0)),
            pl.BlockSpec((1, 4 * DIM), lambda s: (0, 0)),
            pl.BlockSpec((1, DIM), lambda s: (0, 0)),
            pl.BlockSpec((1, DIM), lambda s: (0, 0)),
        ],
        out_specs=pl.BlockSpec((MB, 4 * DIM), lambda s: (s, 0)),
        out_shape=jax.ShapeDtypeStruct((tokens, 4 * DIM), jnp.float32),
        compiler_params=pltpu.CompilerParams(
            dimension_semantics=("parallel",),
            vmem_limit_bytes=100 * 1024 * 1024,
        ),
    )(xf, expand_w, expand_b.reshape(1, -1), pe_norm_g.reshape(1, -1),
      pe_norm_b.reshape(1, -1))
    xs = y.reshape(B, h, w, 2, 2, DIM).transpose(0, 1, 3, 2, 4, 5).reshape(B, H, W, DIM)

    pad_mask = np.zeros((NPAD, NPAD), np.float32)
    pad_mask[:, N:] = NEG
    rb0 = jnp.transpose(rel_bias[0][_REL_IDX], (2, 0, 1))
    rb1 = jnp.transpose(rel_bias[1][_REL_IDX], (2, 0, 1))
    rbp0 = jnp.pad(rb0, ((0, 0), (0, NPAD - N), (0, NPAD - N))) + pad_mask
    rbp1 = jnp.pad(rb1, ((0, 0), (0, NPAD - N), (0, NPAD - N))) + pad_mask
    bias0 = jnp.broadcast_to(rbp0[None], (WIN_PER_STEP, HEADS, NPAD, NPAD))
    smask = jnp.pad(jnp.asarray(_SHIFT_MASK), ((0, 0), (0, NPAD - N), (0, NPAD - N)))
    bias1 = rbp1[None] + smask[:, None]

    args1 = (norm1_g[0].reshape(1, -1), norm1_b[0].reshape(1, -1), qkv_w[0],
             qkv_b[0].reshape(1, -1), proj_w[0], proj_b[0].reshape(1, -1),
             norm2_g[0].reshape(1, -1), norm2_b[0].reshape(1, -1), mlp_w1[0],
             mlp_b1[0].reshape(1, -1), mlp_w2[0], mlp_b2[0].reshape(1, -1))
    args2 = (norm1_g[1].reshape(1, -1), norm1_b[1].reshape(1, -1), qkv_w[1],
             qkv_b[1].reshape(1, -1), proj_w[1], proj_b[1].reshape(1, -1),
             norm2_g[1].reshape(1, -1), norm2_b[1].reshape(1, -1), mlp_w1[1],
             mlp_b1[1].reshape(1, -1), mlp_w2[1], mlp_b2[1].reshape(1, -1))

    xw = _windows_pad(xs)
    xw = _swin_block(xw, bias0, *args1)
    x1 = _unwindows(xw, B, H, W)

    x1r = jnp.roll(x1, (-(WS // 2), -(WS // 2)), axis=(1, 2))
    xw2 = _windows_pad(x1r)
    xw2 = _swin_block(xw2, bias1, *args2)
    x2r = _unwindows(xw2, B, H, W)
    return jnp.roll(x2r, (WS // 2, WS // 2), axis=(1, 2))
